# baseline (speedup 1.0000x reference)
import functools

import jax
import jax.numpy as jnp
import numpy as np
from jax import lax
from jax.experimental import pallas as pl
from jax.experimental.pallas import tpu as pltpu
from jax.experimental.pallas import tpu_sc as plsc

NCLS = 19
RPS = 8
CPAD = 24

SUMOFF = 0
CNTOFF = NCLS * 256
SQOFF = CNTOFF + 128
LSTRIDE = SQOFF + 128
ACCW = 16 * LSTRIDE


def _x_weight_matrix(in_w, out_w):
    xs = np.linspace(0.0, in_w - 1.0, out_w)
    x0 = np.floor(xs).astype(np.int64)
    x1 = np.minimum(x0 + 1, in_w - 1)
    wx = xs - np.floor(xs)
    wm = np.zeros((in_w, out_w), np.float64)
    np.add.at(wm, (x0, np.arange(out_w)), 1.0 - wx)
    np.add.at(wm, (x1, np.arange(out_w)), wx)
    return wm.astype(np.float32)


def _labels_body(wx_ref, s_sm, t_sm, s_out, t_out, s_scr, t_scr, sem,
                 *, in_h, out_h, out_w, nb):
    b = pl.program_id(0)
    t = pl.program_id(1)
    nsteps = pl.num_programs(0) * nb
    step = b * nb + t
    first = step == 0
    num = in_h - 1
    den = out_h - 1

    def row_dmas(bb, tt, buf, sm_ref, scr_ref, dom):
        copies = []
        for r in range(RPS):
            i = tt * RPS + r
            y0 = (i * num) // den
            y1 = jnp.minimum(y0 + 1, in_h - 1)
            copies.append(pltpu.make_async_copy(
                sm_ref.at[bb, :, y0, :], scr_ref.at[buf, 2 * r],
                sem.at[buf, dom]))
            copies.append(pltpu.make_async_copy(
                sm_ref.at[bb, :, y1, :], scr_ref.at[buf, 2 * r + 1],
                sem.at[buf, dom]))
        return copies

    def issue(bb, tt, buf):
        for c in row_dmas(bb, tt, buf, s_sm, s_scr, 0):
            c.start()
        for c in row_dmas(bb, tt, buf, t_sm, t_scr, 1):
            c.start()

    def drain(bb, tt, buf):
        for c in row_dmas(bb, tt, buf, s_sm, s_scr, 0):
            c.wait()
        for c in row_dmas(bb, tt, buf, t_sm, t_scr, 1):
            c.wait()

    cur = lax.rem(step, 2)

    @pl.when(first)
    def _prologue():
        issue(b, t, cur)

    @pl.when(step < nsteps - 1)
    def _prefetch():
        nstep = step + 1
        bn = nstep // nb
        tn = lax.rem(nstep, nb)
        issue(bn, tn, lax.rem(nstep, 2))

    drain(b, t, cur)

    zpad = jnp.zeros((CPAD - NCLS, s_scr.shape[-1]), jnp.float32)
    wmat = wx_ref[...]
    for scr_ref, out_ref in ((s_scr, s_out), (t_scr, t_out)):
        pieces = []
        for r in range(RPS):
            i = t * RPS + r
            y0 = (i * num) // den
            rem = i * num - y0 * den
            wy = rem.astype(jnp.float32) / float(den)
            top = scr_ref[cur, 2 * r]
            bot = scr_ref[cur, 2 * r + 1]
            pieces.append(top * (1.0 - wy) + bot * wy)
            pieces.append(zpad)
        stacked = jnp.concatenate(pieces, axis=0)
        vals = jax.lax.dot_general(
            stacked, wmat,
            dimension_numbers=(((1,), (0,)), ((), ())),
            preferred_element_type=jnp.float32,
            precision=jax.lax.Precision.DEFAULT,
        )
        idx_pieces = []
        for r in range(RPS):
            blk = lax.slice(vals, (CPAD * r, 0), (CPAD * r + CPAD, out_w))
            sub = lax.broadcasted_iota(jnp.int32, (CPAD, out_w), 0)
            real = sub < NCLS
            m = jnp.max(jnp.where(real, blk, -1.0), axis=0, keepdims=True)
            idx = jnp.min(jnp.where((blk >= m) & real, sub, NCLS), axis=0,
                          keepdims=True)
            idx_pieces.append(idx)
        out_ref[...] = jnp.concatenate(idx_pieces, axis=0).reshape(
            out_ref.shape)


def _compute_labels(s_sm, t_sm, out_h, out_w):
    B, C, in_h, in_w = s_sm.shape
    nb = out_h // RPS
    wmat = jnp.asarray(_x_weight_matrix(in_w, out_w))
    any_spec = pl.BlockSpec(memory_space=pl.ANY)
    out_spec = pl.BlockSpec((1, RPS, out_w), lambda b, t: (b, t, 0))
    return pl.pallas_call(
        functools.partial(_labels_body, in_h=in_h, out_h=out_h, out_w=out_w,
                          nb=nb),
        grid=(B, nb),
        in_specs=[pl.BlockSpec((in_w, out_w), lambda b, t: (0, 0)),
                  any_spec, any_spec],
        out_specs=[out_spec, out_spec],
        out_shape=[jax.ShapeDtypeStruct((B, out_h, out_w), jnp.int32),
                   jax.ShapeDtypeStruct((B, out_h, out_w), jnp.int32)],
        scratch_shapes=[
            pltpu.VMEM((2, 2 * RPS, C, in_w), jnp.float32),
            pltpu.VMEM((2, 2 * RPS, C, in_w), jnp.float32),
            pltpu.SemaphoreType.DMA((2, 2)),
        ],
    )(wmat, s_sm, t_sm)


def _sc_stats(s_feat, t_feat, s_lab, t_lab):
    B, F, h, w = s_feat.shape
    nfblk = F // 16
    rows_per = (B * h) // 32 // B

    mesh = plsc.VectorSubcoreMesh(core_axis_name="c", subcore_axis_name="s")

    @functools.partial(
        pl.kernel, mesh=mesh,
        compiler_params=pltpu.CompilerParams(needs_layout_passes=False),
        out_type=jax.ShapeDtypeStruct((64, ACCW), jnp.float32),
        scratch_types=[
            pltpu.VMEM((ACCW,), jnp.float32),
            pltpu.VMEM((16, rows_per, w), jnp.float32),
            pltpu.VMEM((rows_per, w), jnp.int32),
            pltpu.VMEM((rows_per * w // 16 * 16,), jnp.float32),
        ],
    )
    def sc_stats(sf, tf, sl, tl, out, acc, fbuf, lbuf, sqflat):
        wid = lax.axis_index("s") * 2 + lax.axis_index("c")
        r0 = wid * rows_per
        lane = lax.iota(jnp.int32, 16)
        zeros16 = jnp.zeros((16,), jnp.float32)
        ones16 = jnp.ones((16,), jnp.float32)
        ngrp = rows_per * w // 16

        def zero_acc():
            def zbody(i, _):
                acc[pl.ds(i * 16, 16)] = zeros16
                return 0
            lax.fori_loop(0, ACCW // 16, zbody, 0)

        def zero_sq():
            def zbody(i, _):
                sqflat[pl.ds(i * 16, 16)] = zeros16
                return 0
            lax.fori_loop(0, ngrp, zbody, 0)

        for dom, (feat, labs) in enumerate(((sf, sl), (tf, tl))):
            zero_acc()

            def b_body(b, _, feat=feat, labs=labs):
                pltpu.sync_copy(labs.at[b, pl.ds(r0, rows_per), :], lbuf)
                zero_sq()

                def fblk_body(fblk, _):
                    pltpu.sync_copy(
                        feat.at[b, pl.ds(fblk * 16, 16), pl.ds(r0, rows_per), :],
                        fbuf)
                    for rr in range(rows_per):
                        def xg_body(xg, _, rr=rr, fblk_=fblk):
                            lab16 = lbuf[rr, pl.ds(xg * 16, 16)]
                            base = (lane * LSTRIDE + lab16 * 256
                                    + fblk_ * 16)
                            sq = zeros16
                            for f in range(16):
                                v = fbuf[f, rr, pl.ds(xg * 16, 16)]
                                plsc.addupdate_scatter(acc, [base + f], v)
                                sq = sq + v * v
                            sqidx = (rr * 16 + xg) * 16 + lane
                            plsc.addupdate_scatter(sqflat, [sqidx], sq)
                            return 0
                        lax.fori_loop(0, w // 16, xg_body, 0)
                    return 0

                lax.fori_loop(0, nfblk, fblk_body, 0)

                for rr in range(rows_per):
                    def dr_body(xg, _, rr=rr):
                        lab16 = lbuf[rr, pl.ds(xg * 16, 16)]
                        sqv = sqflat[pl.ds((rr * 16 + xg) * 16, 16)]
                        plsc.addupdate_scatter(
                            acc, [lane * LSTRIDE + SQOFF + lab16], sqv)
                        plsc.addupdate_scatter(
                            acc, [lane * LSTRIDE + CNTOFF + lab16], ones16)
                        return 0
                    lax.fori_loop(0, w // 16, dr_body, 0)
                return 0

            lax.fori_loop(0, B, b_body, 0)
            pltpu.sync_copy(acc, out.at[dom * 32 + wid])

    return sc_stats(s_feat, t_feat, s_lab, t_lab)


def _reduce_body(acc_ref, out_ref, *, fdim):
    fdim_f = float(fdim)
    accs = acc_ref[...]
    nrows = accs.shape[0] // 2

    def half(lo):
        red = jnp.sum(lax.slice(accs, (lo, 0), (lo + nrows, ACCW)),
                      axis=0, keepdims=True)
        psum = None
        for l in range(16):
            piece = lax.slice(red, (0, l * LSTRIDE), (1, (l + 1) * LSTRIDE))
            psum = piece if psum is None else psum + piece
        return psum

    ps = half(0)
    pt = half(nrows)

    def stats(p):
        sums = jnp.concatenate(
            [lax.slice(p, (0, c * fdim), (1, (c + 1) * fdim))
             for c in range(NCLS)], axis=0)
        iota_s = lax.broadcasted_iota(jnp.int32, (NCLS, 1), 0)
        cnt_col = jnp.zeros((NCLS, 1), jnp.float32)
        sq_col = jnp.zeros((NCLS, 1), jnp.float32)
        cnt_row = jnp.zeros((1, NCLS), jnp.float32)
        iota_r = lax.broadcasted_iota(jnp.int32, (1, NCLS), 1)
        for c in range(NCLS):
            cv = p[0, CNTOFF + c]
            sv = p[0, SQOFF + c]
            cnt_col = cnt_col + jnp.where(iota_s == c, cv, 0.0)
            sq_col = sq_col + jnp.where(iota_s == c, sv, 0.0)
            cnt_row = cnt_row + jnp.where(iota_r == c, cv, 0.0)
        return sums, cnt_col, sq_col, cnt_row

    sum_s, cnt_s, sq_s, cntrow_s = stats(ps)
    sum_t, cnt_t, sq_t, cntrow_t = stats(pt)

    cnt_tot = cnt_s + cnt_t
    cntrow_tot = cntrow_s + cntrow_t
    valid = cnt_tot > 0.0
    validrow = cntrow_tot > 0.0
    cent = (sum_s + sum_t) / jnp.maximum(cnt_tot, 1.0)
    cn2 = jnp.sum(cent * cent, axis=1, keepdims=True)

    def f2c(sum_d, sq_d, cnt_d):
        dot_cs = jnp.sum(cent * sum_d, axis=1, keepdims=True)
        ssq = jnp.maximum(sq_d - 2.0 * dot_cs + cnt_d * cn2, 0.0)
        ok = cnt_d > 0.0
        nrm = jnp.sqrt(jnp.where(ok, ssq, 1.0))
        dist = nrm / jnp.maximum(cnt_d * fdim_f, 1.0)
        nseen = jnp.sum(jnp.where(ok, 1.0, 0.0))
        return jnp.sum(jnp.where(ok, dist, 0.0)) / jnp.maximum(nseen, 1.0)

    loss_s = f2c(sum_s, sq_s, cnt_s)
    loss_t = f2c(sum_t, sq_t, cnt_t)

    centv = jnp.where(valid, cent, 0.0)
    G = jax.lax.dot_general(
        centv, centv,
        dimension_numbers=(((1,), (1,)), ((), ())),
        preferred_element_type=jnp.float32,
        precision=jax.lax.Precision.HIGHEST,
    )
    ii = lax.broadcasted_iota(jnp.int32, (NCLS, NCLS), 0)
    jj = lax.broadcasted_iota(jnp.int32, (NCLS, NCLS), 1)
    n2col = jnp.sum(centv * centv, axis=1, keepdims=True)
    n2row = jnp.sum(jnp.where(ii == jj, G, 0.0), axis=0, keepdims=True)
    sq = n2col + n2row - 2.0 * G
    pair = (ii != jj) & valid & validrow
    ssq_i = jnp.sum(jnp.where(pair, sq, 0.0), axis=1, keepdims=True)
    nvalid = jnp.sum(jnp.where(valid, 1.0, 0.0))
    denom = jnp.maximum((nvalid - 1.0) * fdim_f, 1.0)
    nrm_i = jnp.sqrt(jnp.where(valid, ssq_i, 1.0))
    dist_i = nrm_i / denom
    c2c = jnp.sum(jnp.where(valid, dist_i, 0.0)) / jnp.maximum(nvalid, 1.0)

    out_ref[...] = jnp.broadcast_to(loss_s + loss_t + c2c, (1, 1))


def kernel(source_feat, source_softmax, target_feat, target_softmax):
    B, F, h, w = source_feat.shape
    s_lab, t_lab = _compute_labels(source_softmax, target_softmax, h, w)
    accs = _sc_stats(source_feat, target_feat, s_lab, t_lab)
    loss = pl.pallas_call(
        functools.partial(_reduce_body, fdim=F),
        in_specs=[pl.BlockSpec((64, ACCW), lambda: (0, 0))],
        out_specs=pl.BlockSpec((1, 1), lambda: (0, 0)),
        out_shape=jax.ShapeDtypeStruct((1, 1), jnp.float32),
    )(accs)
    return loss[0, 0]


# speedup vs baseline: 10.4794x; 10.4794x over previous
import functools

import jax
import jax.numpy as jnp
import numpy as np
from jax import lax
from jax.experimental import pallas as pl
from jax.experimental.pallas import tpu as pltpu

NCLS = 19
RPS = 16
CPAD = 24


def _x_weight_matrix(in_w, out_w):
    xs = np.linspace(0.0, in_w - 1.0, out_w)
    x0 = np.floor(xs).astype(np.int64)
    x1 = np.minimum(x0 + 1, in_w - 1)
    wx = xs - np.floor(xs)
    wm = np.zeros((in_w, out_w), np.float64)
    np.add.at(wm, (x0, np.arange(out_w)), 1.0 - wx)
    np.add.at(wm, (x1, np.arange(out_w)), wx)
    return wm.astype(np.float32)


def _fused_body(wx_ref, s_sm, t_sm, s_feat, t_feat, out_ref,
                s_scr, t_scr, acc_sum_s, acc_misc_s, acc_sum_t, acc_misc_t,
                sem, *, in_h, out_h, out_w, nb, fdim):
    b = pl.program_id(0)
    t = pl.program_id(1)
    nsteps = pl.num_programs(0) * nb
    step = b * nb + t
    first = step == 0
    last = step == nsteps - 1
    num = in_h - 1
    den = out_h - 1
    pb = RPS * out_w

    def row_dmas(bb, tt, buf, sm_ref, scr_ref, dom):
        copies = []
        for r in range(RPS):
            i = tt * RPS + r
            y0 = (i * num) // den
            y1 = jnp.minimum(y0 + 1, in_h - 1)
            copies.append(pltpu.make_async_copy(
                sm_ref.at[bb, :, y0, :], scr_ref.at[buf, 2 * r],
                sem.at[buf, dom]))
            copies.append(pltpu.make_async_copy(
                sm_ref.at[bb, :, y1, :], scr_ref.at[buf, 2 * r + 1],
                sem.at[buf, dom]))
        return copies

    def issue(bb, tt, buf):
        for c in row_dmas(bb, tt, buf, s_sm, s_scr, 0):
            c.start()
        for c in row_dmas(bb, tt, buf, t_sm, t_scr, 1):
            c.start()

    def drain(bb, tt, buf):
        for c in row_dmas(bb, tt, buf, s_sm, s_scr, 0):
            c.wait()
        for c in row_dmas(bb, tt, buf, t_sm, t_scr, 1):
            c.wait()

    cur = lax.rem(step, 2)

    @pl.when(first)
    def _prologue():
        issue(b, t, cur)

    @pl.when(step < nsteps - 1)
    def _prefetch():
        nstep = step + 1
        bn = nstep // nb
        tn = lax.rem(nstep, nb)
        issue(bn, tn, lax.rem(nstep, 2))

    drain(b, t, cur)

    @pl.when(first)
    def _init():
        acc_sum_s[...] = jnp.zeros_like(acc_sum_s)
        acc_misc_s[...] = jnp.zeros_like(acc_misc_s)
        acc_sum_t[...] = jnp.zeros_like(acc_sum_t)
        acc_misc_t[...] = jnp.zeros_like(acc_misc_t)

    zpad = jnp.zeros((CPAD - NCLS, s_scr.shape[-1]), jnp.float32)
    wmat = wx_ref[...]
    for scr_ref, feat_ref, acc_sum, acc_misc in (
            (s_scr, s_feat, acc_sum_s, acc_misc_s),
            (t_scr, t_feat, acc_sum_t, acc_misc_t)):
        pieces = []
        for r in range(RPS):
            i = t * RPS + r
            y0 = (i * num) // den
            rem = i * num - y0 * den
            wy = rem.astype(jnp.float32) / float(den)
            top = scr_ref[cur, 2 * r]
            bot = scr_ref[cur, 2 * r + 1]
            pieces.append(top * (1.0 - wy) + bot * wy)
            pieces.append(zpad)
        stacked = jnp.concatenate(pieces, axis=0)
        vals = jax.lax.dot_general(
            stacked, wmat,
            dimension_numbers=(((1,), (0,)), ((), ())),
            preferred_element_type=jnp.float32,
            precision=jax.lax.Precision.DEFAULT,
        )
        oh_pieces = []
        for r in range(RPS):
            blk = lax.slice(vals, (CPAD * r, 0), (CPAD * r + CPAD, out_w))
            sub = lax.broadcasted_iota(jnp.int32, (CPAD, out_w), 0)
            real = sub < NCLS
            m = jnp.max(jnp.where(real, blk, -1.0), axis=0, keepdims=True)
            idx = jnp.min(jnp.where((blk >= m) & real, sub, NCLS), axis=0,
                          keepdims=True)
            ohr = (lax.broadcasted_iota(jnp.int32, (NCLS, out_w), 0)
                   == idx).astype(jnp.float32)
            oh_pieces.append(ohr)
        ohT = jnp.concatenate(oh_pieces, axis=1)

        feat4 = feat_ref[0]
        feat2 = feat4.reshape(feat4.shape[0], pb)
        acc_sum[...] += jax.lax.dot_general(
            feat2, ohT,
            dimension_numbers=(((1,), (1,)), ((), ())),
            preferred_element_type=jnp.float32,
            precision=jax.lax.Precision.DEFAULT,
        )
        csq8 = jnp.sum(feat4 * feat4, axis=0)
        colsq = csq8.reshape(1, pb)
        ios = lax.broadcasted_iota(jnp.int32, (8, pb), 0)
        extra = jnp.where(ios == 0, colsq,
                          jnp.where(ios == 1, 1.0, 0.0))
        acc_misc[...] += jax.lax.dot_general(
            extra, ohT,
            dimension_numbers=(((1,), (1,)), ((), ())),
            preferred_element_type=jnp.float32,
            precision=jax.lax.Precision.DEFAULT,
        )

    @pl.when(last)
    def _finish():
        fdim_f = float(fdim)
        io8 = lax.broadcasted_iota(jnp.int32, (8, NCLS), 0)

        def row(m_ref, r):
            return jnp.sum(jnp.where(io8 == r, m_ref[...], 0.0), axis=0,
                           keepdims=True)

        sum_s = acc_sum_s[...]
        sum_t = acc_sum_t[...]
        sumsq_s = row(acc_misc_s, 0)
        cnt_s = row(acc_misc_s, 1)
        sumsq_t = row(acc_misc_t, 0)
        cnt_t = row(acc_misc_t, 1)

        cnt_tot = cnt_s + cnt_t
        valid = cnt_tot > 0.0
        cent = (sum_s + sum_t) / jnp.maximum(cnt_tot, 1.0)
        cn2 = jnp.sum(cent * cent, axis=0, keepdims=True)

        def f2c(sum_d, sumsq_d, cnt_d):
            dot_cs = jnp.sum(cent * sum_d, axis=0, keepdims=True)
            ssq = jnp.maximum(sumsq_d - 2.0 * dot_cs + cnt_d * cn2, 0.0)
            ok = cnt_d > 0.0
            nrm = jnp.sqrt(jnp.where(ok, ssq, 1.0))
            dist = nrm / jnp.maximum(cnt_d * fdim_f, 1.0)
            nseen = jnp.sum(jnp.where(ok, 1.0, 0.0))
            return jnp.sum(jnp.where(ok, dist, 0.0)) / jnp.maximum(nseen, 1.0)

        loss_s = f2c(sum_s, sumsq_s, cnt_s)
        loss_t = f2c(sum_t, sumsq_t, cnt_t)

        centv = jnp.where(valid, cent, 0.0)
        n2 = jnp.sum(centv * centv, axis=0, keepdims=True)
        iota_l = lax.broadcasted_iota(jnp.int32, (1, NCLS), 1)
        iota_fl = lax.broadcasted_iota(jnp.int32, (fdim, NCLS), 1)
        ssq_vec = jnp.zeros((1, NCLS), jnp.float32)
        for i in range(NCLS):
            ci = jnp.sum(jnp.where(iota_fl == i, centv, 0.0), axis=1,
                         keepdims=True)
            gi = jnp.sum(ci * centv, axis=0, keepdims=True)
            n2_i = jnp.sum(jnp.where(iota_l == i, n2, 0.0))
            sqrow = n2 + n2_i - 2.0 * gi
            contrib = jnp.sum(jnp.where((iota_l != i) & valid, sqrow, 0.0))
            ssq_vec = ssq_vec + jnp.where(iota_l == i, contrib, 0.0)

        nvalid = jnp.sum(jnp.where(valid, 1.0, 0.0))
        denom = jnp.maximum((nvalid - 1.0) * fdim_f, 1.0)
        nrm_i = jnp.sqrt(jnp.where(valid, ssq_vec, 1.0))
        dist_i = nrm_i / denom
        c2c = jnp.sum(jnp.where(valid, dist_i, 0.0)) / jnp.maximum(nvalid, 1.0)

        out_ref[...] = jnp.broadcast_to(loss_s + loss_t + c2c, (1, 1))


def kernel(source_feat, source_softmax, target_feat, target_softmax):
    B, F, h, w = source_feat.shape
    _, C, in_h, in_w = source_softmax.shape
    nb = h // RPS

    wmat = jnp.asarray(_x_weight_matrix(in_w, w))

    feat_spec = pl.BlockSpec((1, F, RPS, w), lambda b, t: (b, 0, t, 0))
    any_spec = pl.BlockSpec(memory_space=pl.ANY)

    loss = pl.pallas_call(
        functools.partial(_fused_body, in_h=in_h, out_h=h, out_w=w, nb=nb,
                          fdim=F),
        grid=(B, nb),
        in_specs=[pl.BlockSpec((in_w, w), lambda b, t: (0, 0)),
                  any_spec, any_spec, feat_spec, feat_spec],
        out_specs=pl.BlockSpec((1, 1), lambda b, t: (0, 0)),
        out_shape=jax.ShapeDtypeStruct((1, 1), jnp.float32),
        scratch_shapes=[
            pltpu.VMEM((2, 2 * RPS, C, in_w), jnp.float32),
            pltpu.VMEM((2, 2 * RPS, C, in_w), jnp.float32),
            pltpu.VMEM((F, NCLS), jnp.float32),
            pltpu.VMEM((8, NCLS), jnp.float32),
            pltpu.VMEM((F, NCLS), jnp.float32),
            pltpu.VMEM((8, NCLS), jnp.float32),
            pltpu.SemaphoreType.DMA((2, 2)),
        ],
    )(wmat, source_softmax, target_softmax, source_feat, target_feat)
    return loss[0, 0]


# speedup vs baseline: 11.1364x; 1.0627x over previous
import functools

import jax
import jax.numpy as jnp
import numpy as np
from jax import lax
from jax.experimental import pallas as pl
from jax.experimental.pallas import tpu as pltpu

NCLS = 19
RPS = 8
CPAD = 24


def _x_weight_matrix(in_w, out_w):
    xs = np.linspace(0.0, in_w - 1.0, out_w)
    x0 = np.floor(xs).astype(np.int64)
    x1 = np.minimum(x0 + 1, in_w - 1)
    wx = xs - np.floor(xs)
    wm = np.zeros((in_w, out_w), np.float64)
    np.add.at(wm, (x0, np.arange(out_w)), 1.0 - wx)
    np.add.at(wm, (x1, np.arange(out_w)), wx)
    return wm.astype(np.float32)


def _fused_body(wx_ref, s_sm, t_sm, s_feat, t_feat, out_ref,
                s_scr, t_scr, acc_sum_s, acc_misc_s, acc_sum_t, acc_misc_t,
                sem, *, in_h, out_h, out_w, nb, fdim):
    b = pl.program_id(0)
    t = pl.program_id(1)
    nsteps = pl.num_programs(0) * nb
    step = b * nb + t
    first = step == 0
    last = step == nsteps - 1
    num = in_h - 1
    den = out_h - 1
    pb = RPS * out_w

    def row_dmas(bb, tt, buf, sm_ref, scr_ref, dom):
        copies = []
        for r in range(RPS):
            i = tt * RPS + r
            y0 = (i * num) // den
            y1 = jnp.minimum(y0 + 1, in_h - 1)
            copies.append(pltpu.make_async_copy(
                sm_ref.at[bb, :, y0, :], scr_ref.at[buf, 2 * r],
                sem.at[buf, dom]))
            copies.append(pltpu.make_async_copy(
                sm_ref.at[bb, :, y1, :], scr_ref.at[buf, 2 * r + 1],
                sem.at[buf, dom]))
        return copies

    def issue(bb, tt, buf):
        for c in row_dmas(bb, tt, buf, s_sm, s_scr, 0):
            c.start()
        for c in row_dmas(bb, tt, buf, t_sm, t_scr, 1):
            c.start()

    def drain(bb, tt, buf):
        for c in row_dmas(bb, tt, buf, s_sm, s_scr, 0):
            c.wait()
        for c in row_dmas(bb, tt, buf, t_sm, t_scr, 1):
            c.wait()

    cur = lax.rem(step, 2)

    @pl.when(first)
    def _prologue():
        issue(b, t, cur)

    @pl.when(step < nsteps - 1)
    def _prefetch():
        nstep = step + 1
        bn = nstep // nb
        tn = lax.rem(nstep, nb)
        issue(bn, tn, lax.rem(nstep, 2))

    drain(b, t, cur)

    @pl.when(first)
    def _init():
        acc_sum_s[...] = jnp.zeros_like(acc_sum_s)
        acc_misc_s[...] = jnp.zeros_like(acc_misc_s)
        acc_sum_t[...] = jnp.zeros_like(acc_sum_t)
        acc_misc_t[...] = jnp.zeros_like(acc_misc_t)

    zpad = jnp.zeros((CPAD - NCLS, s_scr.shape[-1]), jnp.float32)
    wmat = wx_ref[...]
    for scr_ref, feat_ref, acc_sum, acc_misc in (
            (s_scr, s_feat, acc_sum_s, acc_misc_s),
            (t_scr, t_feat, acc_sum_t, acc_misc_t)):
        pieces = []
        for r in range(RPS):
            i = t * RPS + r
            y0 = (i * num) // den
            rem = i * num - y0 * den
            wy = rem.astype(jnp.float32) / float(den)
            top = scr_ref[cur, 2 * r]
            bot = scr_ref[cur, 2 * r + 1]
            pieces.append(top * (1.0 - wy) + bot * wy)
            pieces.append(zpad)
        stacked = jnp.concatenate(pieces, axis=0)
        vals = jax.lax.dot_general(
            stacked, wmat,
            dimension_numbers=(((1,), (0,)), ((), ())),
            preferred_element_type=jnp.float32,
            precision=jax.lax.Precision.DEFAULT,
        )
        oh_pieces = []
        for r in range(RPS):
            blk = lax.slice(vals, (CPAD * r, 0), (CPAD * r + CPAD, out_w))
            sub = lax.broadcasted_iota(jnp.int32, (CPAD, out_w), 0)
            real = sub < NCLS
            m = jnp.max(jnp.where(real, blk, -1.0), axis=0, keepdims=True)
            idx = jnp.min(jnp.where((blk >= m) & real, sub, NCLS), axis=0,
                          keepdims=True)
            ohr = (lax.broadcasted_iota(jnp.int32, (NCLS, out_w), 0)
                   == idx).astype(jnp.float32)
            oh_pieces.append(ohr)
        ohT = jnp.concatenate(oh_pieces, axis=1)

        feat4 = feat_ref[0]
        feat2 = feat4.reshape(feat4.shape[0], pb)
        acc_sum[...] += jax.lax.dot_general(
            feat2, ohT,
            dimension_numbers=(((1,), (1,)), ((), ())),
            preferred_element_type=jnp.float32,
            precision=jax.lax.Precision.DEFAULT,
        )
        csq8 = jnp.sum(feat4 * feat4, axis=0)
        colsq = csq8.reshape(1, pb)
        ios = lax.broadcasted_iota(jnp.int32, (8, pb), 0)
        extra = jnp.where(ios == 0, colsq,
                          jnp.where(ios == 1, 1.0, 0.0))
        acc_misc[...] += jax.lax.dot_general(
            extra, ohT,
            dimension_numbers=(((1,), (1,)), ((), ())),
            preferred_element_type=jnp.float32,
            precision=jax.lax.Precision.DEFAULT,
        )

    @pl.when(last)
    def _finish():
        fdim_f = float(fdim)
        io8 = lax.broadcasted_iota(jnp.int32, (8, NCLS), 0)

        def row(m_ref, r):
            return jnp.sum(jnp.where(io8 == r, m_ref[...], 0.0), axis=0,
                           keepdims=True)

        sum_s = acc_sum_s[...]
        sum_t = acc_sum_t[...]
        sumsq_s = row(acc_misc_s, 0)
        cnt_s = row(acc_misc_s, 1)
        sumsq_t = row(acc_misc_t, 0)
        cnt_t = row(acc_misc_t, 1)

        cnt_tot = cnt_s + cnt_t
        valid = cnt_tot > 0.0
        cent = (sum_s + sum_t) / jnp.maximum(cnt_tot, 1.0)
        cn2 = jnp.sum(cent * cent, axis=0, keepdims=True)

        def f2c(sum_d, sumsq_d, cnt_d):
            dot_cs = jnp.sum(cent * sum_d, axis=0, keepdims=True)
            ssq = jnp.maximum(sumsq_d - 2.0 * dot_cs + cnt_d * cn2, 0.0)
            ok = cnt_d > 0.0
            nrm = jnp.sqrt(jnp.where(ok, ssq, 1.0))
            dist = nrm / jnp.maximum(cnt_d * fdim_f, 1.0)
            nseen = jnp.sum(jnp.where(ok, 1.0, 0.0))
            return jnp.sum(jnp.where(ok, dist, 0.0)) / jnp.maximum(nseen, 1.0)

        loss_s = f2c(sum_s, sumsq_s, cnt_s)
        loss_t = f2c(sum_t, sumsq_t, cnt_t)

        centv = jnp.where(valid, cent, 0.0)
        n2 = jnp.sum(centv * centv, axis=0, keepdims=True)
        iota_l = lax.broadcasted_iota(jnp.int32, (1, NCLS), 1)
        iota_fl = lax.broadcasted_iota(jnp.int32, (fdim, NCLS), 1)
        ssq_vec = jnp.zeros((1, NCLS), jnp.float32)
        for i in range(NCLS):
            ci = jnp.sum(jnp.where(iota_fl == i, centv, 0.0), axis=1,
                         keepdims=True)
            gi = jnp.sum(ci * centv, axis=0, keepdims=True)
            n2_i = jnp.sum(jnp.where(iota_l == i, n2, 0.0))
            sqrow = n2 + n2_i - 2.0 * gi
            contrib = jnp.sum(jnp.where((iota_l != i) & valid, sqrow, 0.0))
            ssq_vec = ssq_vec + jnp.where(iota_l == i, contrib, 0.0)

        nvalid = jnp.sum(jnp.where(valid, 1.0, 0.0))
        denom = jnp.maximum((nvalid - 1.0) * fdim_f, 1.0)
        nrm_i = jnp.sqrt(jnp.where(valid, ssq_vec, 1.0))
        dist_i = nrm_i / denom
        c2c = jnp.sum(jnp.where(valid, dist_i, 0.0)) / jnp.maximum(nvalid, 1.0)

        out_ref[...] = jnp.broadcast_to(loss_s + loss_t + c2c, (1, 1))


def kernel(source_feat, source_softmax, target_feat, target_softmax):
    B, F, h, w = source_feat.shape
    _, C, in_h, in_w = source_softmax.shape
    nb = h // RPS

    wmat = jnp.asarray(_x_weight_matrix(in_w, w))

    feat_spec = pl.BlockSpec((1, F, RPS, w), lambda b, t: (b, 0, t, 0))
    any_spec = pl.BlockSpec(memory_space=pl.ANY)

    loss = pl.pallas_call(
        functools.partial(_fused_body, in_h=in_h, out_h=h, out_w=w, nb=nb,
                          fdim=F),
        grid=(B, nb),
        in_specs=[pl.BlockSpec((in_w, w), lambda b, t: (0, 0)),
                  any_spec, any_spec, feat_spec, feat_spec],
        out_specs=pl.BlockSpec((1, 1), lambda b, t: (0, 0)),
        out_shape=jax.ShapeDtypeStruct((1, 1), jnp.float32),
        scratch_shapes=[
            pltpu.VMEM((2, 2 * RPS, C, in_w), jnp.float32),
            pltpu.VMEM((2, 2 * RPS, C, in_w), jnp.float32),
            pltpu.VMEM((F, NCLS), jnp.float32),
            pltpu.VMEM((8, NCLS), jnp.float32),
            pltpu.VMEM((F, NCLS), jnp.float32),
            pltpu.VMEM((8, NCLS), jnp.float32),
            pltpu.SemaphoreType.DMA((2, 2)),
        ],
    )(wmat, source_softmax, target_softmax, source_feat, target_feat)
    return loss[0, 0]


# speedup vs baseline: 11.2619x; 1.0113x over previous
import functools

import jax
import jax.numpy as jnp
import numpy as np
from jax import lax
from jax.experimental import pallas as pl
from jax.experimental.pallas import tpu as pltpu

NCLS = 19
RPS = 8
CPAD = 24


def _x_weight_matrix(in_w, out_w):
    xs = np.linspace(0.0, in_w - 1.0, out_w)
    x0 = np.floor(xs).astype(np.int64)
    x1 = np.minimum(x0 + 1, in_w - 1)
    wx = xs - np.floor(xs)
    wm = np.zeros((in_w, out_w), np.float64)
    np.add.at(wm, (x0, np.arange(out_w)), 1.0 - wx)
    np.add.at(wm, (x1, np.arange(out_w)), wx)
    return wm.astype(np.float32)


def _fused_body(wx_ref, s_sm, t_sm, s_feat, t_feat, out_ref,
                s_scr, t_scr, acc_sum_s, acc_misc_s, acc_sum_t, acc_misc_t,
                sem, *, in_h, out_h, out_w, nb, fdim):
    b = pl.program_id(0)
    t = pl.program_id(1)
    nsteps = pl.num_programs(0) * nb
    step = b * nb + t
    first = step == 0
    last = step == nsteps - 1
    num = in_h - 1
    den = out_h - 1
    pb = RPS * out_w

    def row_dmas(bb, tt, buf, sm_ref, scr_ref, dom):
        copies = []
        for r in range(RPS):
            i = tt * RPS + r
            y0 = (i * num) // den
            y1 = jnp.minimum(y0 + 1, in_h - 1)
            copies.append(pltpu.make_async_copy(
                sm_ref.at[bb, :, y0, :], scr_ref.at[buf, 2 * r],
                sem.at[buf, dom]))
            copies.append(pltpu.make_async_copy(
                sm_ref.at[bb, :, y1, :], scr_ref.at[buf, 2 * r + 1],
                sem.at[buf, dom]))
        return copies

    def issue(bb, tt, buf):
        for c in row_dmas(bb, tt, buf, s_sm, s_scr, 0):
            c.start()
        for c in row_dmas(bb, tt, buf, t_sm, t_scr, 1):
            c.start()

    def drain(bb, tt, buf):
        for c in row_dmas(bb, tt, buf, s_sm, s_scr, 0):
            c.wait()
        for c in row_dmas(bb, tt, buf, t_sm, t_scr, 1):
            c.wait()

    cur = lax.rem(step, 3)

    @pl.when(first)
    def _prologue():
        issue(b, t, cur)
        issue(0, 1, 1)

    @pl.when(step + 2 < nsteps)
    def _prefetch():
        nstep = step + 2
        bn = nstep // nb
        tn = lax.rem(nstep, nb)
        issue(bn, tn, lax.rem(nstep, 3))

    drain(b, t, cur)

    @pl.when(first)
    def _init():
        acc_sum_s[...] = jnp.zeros_like(acc_sum_s)
        acc_misc_s[...] = jnp.zeros_like(acc_misc_s)
        acc_sum_t[...] = jnp.zeros_like(acc_sum_t)
        acc_misc_t[...] = jnp.zeros_like(acc_misc_t)

    zpad = jnp.zeros((CPAD - NCLS, s_scr.shape[-1]), jnp.float32)
    wmat = wx_ref[...]
    for scr_ref, feat_ref, acc_sum, acc_misc in (
            (s_scr, s_feat, acc_sum_s, acc_misc_s),
            (t_scr, t_feat, acc_sum_t, acc_misc_t)):
        pieces = []
        for r in range(RPS):
            i = t * RPS + r
            y0 = (i * num) // den
            rem = i * num - y0 * den
            wy = rem.astype(jnp.float32) / float(den)
            top = scr_ref[cur, 2 * r]
            bot = scr_ref[cur, 2 * r + 1]
            pieces.append(top * (1.0 - wy) + bot * wy)
            pieces.append(zpad)
        stacked = jnp.concatenate(pieces, axis=0)
        vals = jax.lax.dot_general(
            stacked, wmat,
            dimension_numbers=(((1,), (0,)), ((), ())),
            preferred_element_type=jnp.float32,
            precision=jax.lax.Precision.DEFAULT,
        )
        oh_pieces = []
        for r in range(RPS):
            blk = lax.slice(vals, (CPAD * r, 0), (CPAD * r + CPAD, out_w))
            sub = lax.broadcasted_iota(jnp.int32, (CPAD, out_w), 0)
            real = sub < NCLS
            m = jnp.max(jnp.where(real, blk, -1.0), axis=0, keepdims=True)
            idx = jnp.min(jnp.where((blk >= m) & real, sub, NCLS), axis=0,
                          keepdims=True)
            ohr = (lax.broadcasted_iota(jnp.int32, (NCLS, out_w), 0)
                   == idx).astype(jnp.float32)
            oh_pieces.append(ohr)
        ohT = jnp.concatenate(oh_pieces, axis=1)

        feat4 = feat_ref[0]
        feat2 = feat4.reshape(feat4.shape[0], pb)
        acc_sum[...] += jax.lax.dot_general(
            feat2, ohT,
            dimension_numbers=(((1,), (1,)), ((), ())),
            preferred_element_type=jnp.float32,
            precision=jax.lax.Precision.DEFAULT,
        )
        csq8 = jnp.sum(feat4 * feat4, axis=0)
        colsq = csq8.reshape(1, pb)
        ios = lax.broadcasted_iota(jnp.int32, (8, pb), 0)
        extra = jnp.where(ios == 0, colsq,
                          jnp.where(ios == 1, 1.0, 0.0))
        acc_misc[...] += jax.lax.dot_general(
            extra, ohT,
            dimension_numbers=(((1,), (1,)), ((), ())),
            preferred_element_type=jnp.float32,
            precision=jax.lax.Precision.DEFAULT,
        )

    @pl.when(last)
    def _finish():
        fdim_f = float(fdim)
        io8 = lax.broadcasted_iota(jnp.int32, (8, NCLS), 0)

        def row(m_ref, r):
            return jnp.sum(jnp.where(io8 == r, m_ref[...], 0.0), axis=0,
                           keepdims=True)

        sum_s = acc_sum_s[...]
        sum_t = acc_sum_t[...]
        sumsq_s = row(acc_misc_s, 0)
        cnt_s = row(acc_misc_s, 1)
        sumsq_t = row(acc_misc_t, 0)
        cnt_t = row(acc_misc_t, 1)

        cnt_tot = cnt_s + cnt_t
        valid = cnt_tot > 0.0
        cent = (sum_s + sum_t) / jnp.maximum(cnt_tot, 1.0)
        cn2 = jnp.sum(cent * cent, axis=0, keepdims=True)

        def f2c(sum_d, sumsq_d, cnt_d):
            dot_cs = jnp.sum(cent * sum_d, axis=0, keepdims=True)
            ssq = jnp.maximum(sumsq_d - 2.0 * dot_cs + cnt_d * cn2, 0.0)
            ok = cnt_d > 0.0
            nrm = jnp.sqrt(jnp.where(ok, ssq, 1.0))
            dist = nrm / jnp.maximum(cnt_d * fdim_f, 1.0)
            nseen = jnp.sum(jnp.where(ok, 1.0, 0.0))
            return jnp.sum(jnp.where(ok, dist, 0.0)) / jnp.maximum(nseen, 1.0)

        loss_s = f2c(sum_s, sumsq_s, cnt_s)
        loss_t = f2c(sum_t, sumsq_t, cnt_t)

        centv = jnp.where(valid, cent, 0.0)
        n2 = jnp.sum(centv * centv, axis=0, keepdims=True)
        iota_l = lax.broadcasted_iota(jnp.int32, (1, NCLS), 1)
        iota_fl = lax.broadcasted_iota(jnp.int32, (fdim, NCLS), 1)
        ssq_vec = jnp.zeros((1, NCLS), jnp.float32)
        for i in range(NCLS):
            ci = jnp.sum(jnp.where(iota_fl == i, centv, 0.0), axis=1,
                         keepdims=True)
            gi = jnp.sum(ci * centv, axis=0, keepdims=True)
            n2_i = jnp.sum(jnp.where(iota_l == i, n2, 0.0))
            sqrow = n2 + n2_i - 2.0 * gi
            contrib = jnp.sum(jnp.where((iota_l != i) & valid, sqrow, 0.0))
            ssq_vec = ssq_vec + jnp.where(iota_l == i, contrib, 0.0)

        nvalid = jnp.sum(jnp.where(valid, 1.0, 0.0))
        denom = jnp.maximum((nvalid - 1.0) * fdim_f, 1.0)
        nrm_i = jnp.sqrt(jnp.where(valid, ssq_vec, 1.0))
        dist_i = nrm_i / denom
        c2c = jnp.sum(jnp.where(valid, dist_i, 0.0)) / jnp.maximum(nvalid, 1.0)

        out_ref[...] = jnp.broadcast_to(loss_s + loss_t + c2c, (1, 1))


def kernel(source_feat, source_softmax, target_feat, target_softmax):
    B, F, h, w = source_feat.shape
    _, C, in_h, in_w = source_softmax.shape
    nb = h // RPS

    wmat = jnp.asarray(_x_weight_matrix(in_w, w))

    feat_spec = pl.BlockSpec((1, F, RPS, w), lambda b, t: (b, 0, t, 0))
    any_spec = pl.BlockSpec(memory_space=pl.ANY)

    loss = pl.pallas_call(
        functools.partial(_fused_body, in_h=in_h, out_h=h, out_w=w, nb=nb,
                          fdim=F),
        grid=(B, nb),
        in_specs=[pl.BlockSpec((in_w, w), lambda b, t: (0, 0)),
                  any_spec, any_spec, feat_spec, feat_spec],
        out_specs=pl.BlockSpec((1, 1), lambda b, t: (0, 0)),
        out_shape=jax.ShapeDtypeStruct((1, 1), jnp.float32),
        scratch_shapes=[
            pltpu.VMEM((3, 2 * RPS, C, in_w), jnp.float32),
            pltpu.VMEM((3, 2 * RPS, C, in_w), jnp.float32),
            pltpu.VMEM((F, NCLS), jnp.float32),
            pltpu.VMEM((8, NCLS), jnp.float32),
            pltpu.VMEM((F, NCLS), jnp.float32),
            pltpu.VMEM((8, NCLS), jnp.float32),
            pltpu.SemaphoreType.DMA((3, 2)),
        ],
    )(wmat, source_softmax, target_softmax, source_feat, target_feat)
    return loss[0, 0]


# speedup vs baseline: 11.3546x; 1.0082x over previous
import functools

import jax
import jax.numpy as jnp
import numpy as np
from jax import lax
from jax.experimental import pallas as pl
from jax.experimental.pallas import tpu as pltpu

NCLS = 19
RPS = 8
CPAD = 24


def _x_weight_matrix(in_w, out_w):
    xs = np.linspace(0.0, in_w - 1.0, out_w)
    x0 = np.floor(xs).astype(np.int64)
    x1 = np.minimum(x0 + 1, in_w - 1)
    wx = xs - np.floor(xs)
    wm = np.zeros((in_w, out_w), np.float64)
    np.add.at(wm, (x0, np.arange(out_w)), 1.0 - wx)
    np.add.at(wm, (x1, np.arange(out_w)), wx)
    return wm.astype(np.float32)


def _fused_body(wx_ref, s_sm, t_sm, s_feat, t_feat, out_ref,
                s_scr, t_scr, acc_sum_s, acc_misc_s, acc_sum_t, acc_misc_t,
                sem, *, in_h, out_h, out_w, nb, fdim):
    b = pl.program_id(0)
    t = pl.program_id(1)
    nsteps = pl.num_programs(0) * nb
    step = b * nb + t
    first = step == 0
    last = step == nsteps - 1
    num = in_h - 1
    den = out_h - 1
    pb = RPS * out_w

    def row_dmas(bb, tt, buf, sm_ref, scr_ref, dom):
        copies = []
        for r in range(RPS):
            i = tt * RPS + r
            y0 = (i * num) // den
            y1 = jnp.minimum(y0 + 1, in_h - 1)
            copies.append(pltpu.make_async_copy(
                sm_ref.at[bb, :, y0, :], scr_ref.at[buf, 2 * r],
                sem.at[buf, dom]))
            copies.append(pltpu.make_async_copy(
                sm_ref.at[bb, :, y1, :], scr_ref.at[buf, 2 * r + 1],
                sem.at[buf, dom]))
        return copies

    def issue(bb, tt, buf):
        for c in row_dmas(bb, tt, buf, s_sm, s_scr, 0):
            c.start()
        for c in row_dmas(bb, tt, buf, t_sm, t_scr, 1):
            c.start()

    def drain(bb, tt, buf):
        for c in row_dmas(bb, tt, buf, s_sm, s_scr, 0):
            c.wait()
        for c in row_dmas(bb, tt, buf, t_sm, t_scr, 1):
            c.wait()

    cur = lax.rem(step, 4)

    @pl.when(first)
    def _prologue():
        issue(b, t, cur)
        issue(0, 1, 1)
        issue(0, 2, 2)

    @pl.when(step + 3 < nsteps)
    def _prefetch():
        nstep = step + 3
        bn = nstep // nb
        tn = lax.rem(nstep, nb)
        issue(bn, tn, lax.rem(nstep, 4))

    drain(b, t, cur)

    @pl.when(first)
    def _init():
        acc_sum_s[...] = jnp.zeros_like(acc_sum_s)
        acc_misc_s[...] = jnp.zeros_like(acc_misc_s)
        acc_sum_t[...] = jnp.zeros_like(acc_sum_t)
        acc_misc_t[...] = jnp.zeros_like(acc_misc_t)

    zpad = jnp.zeros((CPAD - NCLS, s_scr.shape[-1]), jnp.float32)
    wmat = wx_ref[...]
    for scr_ref, feat_ref, acc_sum, acc_misc in (
            (s_scr, s_feat, acc_sum_s, acc_misc_s),
            (t_scr, t_feat, acc_sum_t, acc_misc_t)):
        pieces = []
        for r in range(RPS):
            i = t * RPS + r
            y0 = (i * num) // den
            rem = i * num - y0 * den
            wy = rem.astype(jnp.float32) / float(den)
            top = scr_ref[cur, 2 * r]
            bot = scr_ref[cur, 2 * r + 1]
            pieces.append(top * (1.0 - wy) + bot * wy)
            pieces.append(zpad)
        stacked = jnp.concatenate(pieces, axis=0)
        vals = jax.lax.dot_general(
            stacked, wmat,
            dimension_numbers=(((1,), (0,)), ((), ())),
            preferred_element_type=jnp.float32,
            precision=jax.lax.Precision.DEFAULT,
        )
        oh_pieces = []
        for r in range(RPS):
            blk = lax.slice(vals, (CPAD * r, 0), (CPAD * r + CPAD, out_w))
            sub = lax.broadcasted_iota(jnp.int32, (CPAD, out_w), 0)
            real = sub < NCLS
            m = jnp.max(jnp.where(real, blk, -1.0), axis=0, keepdims=True)
            idx = jnp.min(jnp.where((blk >= m) & real, sub, NCLS), axis=0,
                          keepdims=True)
            ohr = (lax.broadcasted_iota(jnp.int32, (NCLS, out_w), 0)
                   == idx).astype(jnp.float32)
            oh_pieces.append(ohr)
        ohT = jnp.concatenate(oh_pieces, axis=1)

        feat4 = feat_ref[0]
        feat2 = feat4.reshape(feat4.shape[0], pb)
        acc_sum[...] += jax.lax.dot_general(
            feat2, ohT,
            dimension_numbers=(((1,), (1,)), ((), ())),
            preferred_element_type=jnp.float32,
            precision=jax.lax.Precision.DEFAULT,
        )
        csq8 = jnp.sum(feat4 * feat4, axis=0)
        colsq = csq8.reshape(1, pb)
        ios = lax.broadcasted_iota(jnp.int32, (8, pb), 0)
        extra = jnp.where(ios == 0, colsq,
                          jnp.where(ios == 1, 1.0, 0.0))
        acc_misc[...] += jax.lax.dot_general(
            extra, ohT,
            dimension_numbers=(((1,), (1,)), ((), ())),
            preferred_element_type=jnp.float32,
            precision=jax.lax.Precision.DEFAULT,
        )

    @pl.when(last)
    def _finish():
        fdim_f = float(fdim)
        io8 = lax.broadcasted_iota(jnp.int32, (8, NCLS), 0)

        def row(m_ref, r):
            return jnp.sum(jnp.where(io8 == r, m_ref[...], 0.0), axis=0,
                           keepdims=True)

        sum_s = acc_sum_s[...]
        sum_t = acc_sum_t[...]
        sumsq_s = row(acc_misc_s, 0)
        cnt_s = row(acc_misc_s, 1)
        sumsq_t = row(acc_misc_t, 0)
        cnt_t = row(acc_misc_t, 1)

        cnt_tot = cnt_s + cnt_t
        valid = cnt_tot > 0.0
        cent = (sum_s + sum_t) / jnp.maximum(cnt_tot, 1.0)
        cn2 = jnp.sum(cent * cent, axis=0, keepdims=True)

        def f2c(sum_d, sumsq_d, cnt_d):
            dot_cs = jnp.sum(cent * sum_d, axis=0, keepdims=True)
            ssq = jnp.maximum(sumsq_d - 2.0 * dot_cs + cnt_d * cn2, 0.0)
            ok = cnt_d > 0.0
            nrm = jnp.sqrt(jnp.where(ok, ssq, 1.0))
            dist = nrm / jnp.maximum(cnt_d * fdim_f, 1.0)
            nseen = jnp.sum(jnp.where(ok, 1.0, 0.0))
            return jnp.sum(jnp.where(ok, dist, 0.0)) / jnp.maximum(nseen, 1.0)

        loss_s = f2c(sum_s, sumsq_s, cnt_s)
        loss_t = f2c(sum_t, sumsq_t, cnt_t)

        centv = jnp.where(valid, cent, 0.0)
        n2 = jnp.sum(centv * centv, axis=0, keepdims=True)
        iota_l = lax.broadcasted_iota(jnp.int32, (1, NCLS), 1)
        iota_fl = lax.broadcasted_iota(jnp.int32, (fdim, NCLS), 1)
        ssq_vec = jnp.zeros((1, NCLS), jnp.float32)
        for i in range(NCLS):
            ci = jnp.sum(jnp.where(iota_fl == i, centv, 0.0), axis=1,
                         keepdims=True)
            gi = jnp.sum(ci * centv, axis=0, keepdims=True)
            n2_i = jnp.sum(jnp.where(iota_l == i, n2, 0.0))
            sqrow = n2 + n2_i - 2.0 * gi
            contrib = jnp.sum(jnp.where((iota_l != i) & valid, sqrow, 0.0))
            ssq_vec = ssq_vec + jnp.where(iota_l == i, contrib, 0.0)

        nvalid = jnp.sum(jnp.where(valid, 1.0, 0.0))
        denom = jnp.maximum((nvalid - 1.0) * fdim_f, 1.0)
        nrm_i = jnp.sqrt(jnp.where(valid, ssq_vec, 1.0))
        dist_i = nrm_i / denom
        c2c = jnp.sum(jnp.where(valid, dist_i, 0.0)) / jnp.maximum(nvalid, 1.0)

        out_ref[...] = jnp.broadcast_to(loss_s + loss_t + c2c, (1, 1))


def kernel(source_feat, source_softmax, target_feat, target_softmax):
    B, F, h, w = source_feat.shape
    _, C, in_h, in_w = source_softmax.shape
    nb = h // RPS

    wmat = jnp.asarray(_x_weight_matrix(in_w, w))

    feat_spec = pl.BlockSpec((1, F, RPS, w), lambda b, t: (b, 0, t, 0))
    any_spec = pl.BlockSpec(memory_space=pl.ANY)

    loss = pl.pallas_call(
        functools.partial(_fused_body, in_h=in_h, out_h=h, out_w=w, nb=nb,
                          fdim=F),
        grid=(B, nb),
        in_specs=[pl.BlockSpec((in_w, w), lambda b, t: (0, 0)),
                  any_spec, any_spec, feat_spec, feat_spec],
        out_specs=pl.BlockSpec((1, 1), lambda b, t: (0, 0)),
        out_shape=jax.ShapeDtypeStruct((1, 1), jnp.float32),
        scratch_shapes=[
            pltpu.VMEM((4, 2 * RPS, C, in_w), jnp.float32),
            pltpu.VMEM((4, 2 * RPS, C, in_w), jnp.float32),
            pltpu.VMEM((F, NCLS), jnp.float32),
            pltpu.VMEM((8, NCLS), jnp.float32),
            pltpu.VMEM((F, NCLS), jnp.float32),
            pltpu.VMEM((8, NCLS), jnp.float32),
            pltpu.SemaphoreType.DMA((4, 2)),
        ],
    )(wmat, source_softmax, target_softmax, source_feat, target_feat)
    return loss[0, 0]


# speedup vs baseline: 11.3600x; 1.0005x over previous
import functools

import jax
import jax.numpy as jnp
import numpy as np
from jax import lax
from jax.experimental import pallas as pl
from jax.experimental.pallas import tpu as pltpu

NCLS = 19
RPS = 8
CPAD = 24


def _x_weight_matrix(in_w, out_w):
    xs = np.linspace(0.0, in_w - 1.0, out_w)
    x0 = np.floor(xs).astype(np.int64)
    x1 = np.minimum(x0 + 1, in_w - 1)
    wx = xs - np.floor(xs)
    wm = np.zeros((in_w, out_w), np.float64)
    np.add.at(wm, (x0, np.arange(out_w)), 1.0 - wx)
    np.add.at(wm, (x1, np.arange(out_w)), wx)
    return wm.astype(np.float32)


def _fused_body(wx_ref, s_sm, t_sm, s_feat, t_feat, out_ref,
                s_scr, t_scr, acc_sum_s, acc_misc_s, acc_sum_t, acc_misc_t,
                sem, *, in_h, out_h, out_w, nb, fdim):
    b = pl.program_id(0)
    t = pl.program_id(1)
    nsteps = pl.num_programs(0) * nb
    step = b * nb + t
    first = step == 0
    last = step == nsteps - 1
    num = in_h - 1
    den = out_h - 1
    pb = RPS * out_w

    def row_dmas(bb, tt, buf, sm_ref, scr_ref, dom):
        copies = []
        for r in range(RPS):
            i = tt * RPS + r
            y0 = (i * num) // den
            y1 = jnp.minimum(y0 + 1, in_h - 1)
            copies.append(pltpu.make_async_copy(
                sm_ref.at[bb, :, y0, :], scr_ref.at[buf, 2 * r],
                sem.at[buf, dom]))
            copies.append(pltpu.make_async_copy(
                sm_ref.at[bb, :, y1, :], scr_ref.at[buf, 2 * r + 1],
                sem.at[buf, dom]))
        return copies

    def issue(bb, tt, buf):
        for c in row_dmas(bb, tt, buf, s_sm, s_scr, 0):
            c.start()
        for c in row_dmas(bb, tt, buf, t_sm, t_scr, 1):
            c.start()

    def drain(bb, tt, buf):
        for c in row_dmas(bb, tt, buf, s_sm, s_scr, 0):
            c.wait()
        for c in row_dmas(bb, tt, buf, t_sm, t_scr, 1):
            c.wait()

    cur = lax.rem(step, 5)

    @pl.when(first)
    def _prologue():
        issue(b, t, cur)
        issue(0, 1, 1)
        issue(0, 2, 2)
        issue(0, 3, 3)

    @pl.when(step + 4 < nsteps)
    def _prefetch():
        nstep = step + 4
        bn = nstep // nb
        tn = lax.rem(nstep, nb)
        issue(bn, tn, lax.rem(nstep, 5))

    drain(b, t, cur)

    @pl.when(first)
    def _init():
        acc_sum_s[...] = jnp.zeros_like(acc_sum_s)
        acc_misc_s[...] = jnp.zeros_like(acc_misc_s)
        acc_sum_t[...] = jnp.zeros_like(acc_sum_t)
        acc_misc_t[...] = jnp.zeros_like(acc_misc_t)

    zpad = jnp.zeros((CPAD - NCLS, s_scr.shape[-1]), jnp.float32)
    wmat = wx_ref[...]
    for scr_ref, feat_ref, acc_sum, acc_misc in (
            (s_scr, s_feat, acc_sum_s, acc_misc_s),
            (t_scr, t_feat, acc_sum_t, acc_misc_t)):
        pieces = []
        for r in range(RPS):
            i = t * RPS + r
            y0 = (i * num) // den
            rem = i * num - y0 * den
            wy = rem.astype(jnp.float32) / float(den)
            top = scr_ref[cur, 2 * r]
            bot = scr_ref[cur, 2 * r + 1]
            pieces.append(top * (1.0 - wy) + bot * wy)
            pieces.append(zpad)
        stacked = jnp.concatenate(pieces, axis=0)
        vals = jax.lax.dot_general(
            stacked, wmat,
            dimension_numbers=(((1,), (0,)), ((), ())),
            preferred_element_type=jnp.float32,
            precision=jax.lax.Precision.DEFAULT,
        )
        oh_pieces = []
        for r in range(RPS):
            blk = lax.slice(vals, (CPAD * r, 0), (CPAD * r + CPAD, out_w))
            sub = lax.broadcasted_iota(jnp.int32, (CPAD, out_w), 0)
            real = sub < NCLS
            m = jnp.max(jnp.where(real, blk, -1.0), axis=0, keepdims=True)
            idx = jnp.min(jnp.where((blk >= m) & real, sub, NCLS), axis=0,
                          keepdims=True)
            ohr = (lax.broadcasted_iota(jnp.int32, (NCLS, out_w), 0)
                   == idx).astype(jnp.float32)
            oh_pieces.append(ohr)
        ohT = jnp.concatenate(oh_pieces, axis=1)

        feat4 = feat_ref[0]
        feat2 = feat4.reshape(feat4.shape[0], pb)
        acc_sum[...] += jax.lax.dot_general(
            feat2, ohT,
            dimension_numbers=(((1,), (1,)), ((), ())),
            preferred_element_type=jnp.float32,
            precision=jax.lax.Precision.DEFAULT,
        )
        csq8 = jnp.sum(feat4 * feat4, axis=0)
        colsq = csq8.reshape(1, pb)
        ios = lax.broadcasted_iota(jnp.int32, (8, pb), 0)
        extra = jnp.where(ios == 0, colsq,
                          jnp.where(ios == 1, 1.0, 0.0))
        acc_misc[...] += jax.lax.dot_general(
            extra, ohT,
            dimension_numbers=(((1,), (1,)), ((), ())),
            preferred_element_type=jnp.float32,
            precision=jax.lax.Precision.DEFAULT,
        )

    @pl.when(last)
    def _finish():
        fdim_f = float(fdim)
        io8 = lax.broadcasted_iota(jnp.int32, (8, NCLS), 0)

        def row(m_ref, r):
            return jnp.sum(jnp.where(io8 == r, m_ref[...], 0.0), axis=0,
                           keepdims=True)

        sum_s = acc_sum_s[...]
        sum_t = acc_sum_t[...]
        sumsq_s = row(acc_misc_s, 0)
        cnt_s = row(acc_misc_s, 1)
        sumsq_t = row(acc_misc_t, 0)
        cnt_t = row(acc_misc_t, 1)

        cnt_tot = cnt_s + cnt_t
        valid = cnt_tot > 0.0
        cent = (sum_s + sum_t) / jnp.maximum(cnt_tot, 1.0)
        cn2 = jnp.sum(cent * cent, axis=0, keepdims=True)

        def f2c(sum_d, sumsq_d, cnt_d):
            dot_cs = jnp.sum(cent * sum_d, axis=0, keepdims=True)
            ssq = jnp.maximum(sumsq_d - 2.0 * dot_cs + cnt_d * cn2, 0.0)
            ok = cnt_d > 0.0
            nrm = jnp.sqrt(jnp.where(ok, ssq, 1.0))
            dist = nrm / jnp.maximum(cnt_d * fdim_f, 1.0)
            nseen = jnp.sum(jnp.where(ok, 1.0, 0.0))
            return jnp.sum(jnp.where(ok, dist, 0.0)) / jnp.maximum(nseen, 1.0)

        loss_s = f2c(sum_s, sumsq_s, cnt_s)
        loss_t = f2c(sum_t, sumsq_t, cnt_t)

        centv = jnp.where(valid, cent, 0.0)
        n2 = jnp.sum(centv * centv, axis=0, keepdims=True)
        iota_l = lax.broadcasted_iota(jnp.int32, (1, NCLS), 1)
        iota_fl = lax.broadcasted_iota(jnp.int32, (fdim, NCLS), 1)
        ssq_vec = jnp.zeros((1, NCLS), jnp.float32)
        for i in range(NCLS):
            ci = jnp.sum(jnp.where(iota_fl == i, centv, 0.0), axis=1,
                         keepdims=True)
            gi = jnp.sum(ci * centv, axis=0, keepdims=True)
            n2_i = jnp.sum(jnp.where(iota_l == i, n2, 0.0))
            sqrow = n2 + n2_i - 2.0 * gi
            contrib = jnp.sum(jnp.where((iota_l != i) & valid, sqrow, 0.0))
            ssq_vec = ssq_vec + jnp.where(iota_l == i, contrib, 0.0)

        nvalid = jnp.sum(jnp.where(valid, 1.0, 0.0))
        denom = jnp.maximum((nvalid - 1.0) * fdim_f, 1.0)
        nrm_i = jnp.sqrt(jnp.where(valid, ssq_vec, 1.0))
        dist_i = nrm_i / denom
        c2c = jnp.sum(jnp.where(valid, dist_i, 0.0)) / jnp.maximum(nvalid, 1.0)

        out_ref[...] = jnp.broadcast_to(loss_s + loss_t + c2c, (1, 1))


def kernel(source_feat, source_softmax, target_feat, target_softmax):
    B, F, h, w = source_feat.shape
    _, C, in_h, in_w = source_softmax.shape
    nb = h // RPS

    wmat = jnp.asarray(_x_weight_matrix(in_w, w))

    feat_spec = pl.BlockSpec((1, F, RPS, w), lambda b, t: (b, 0, t, 0))
    any_spec = pl.BlockSpec(memory_space=pl.ANY)

    loss = pl.pallas_call(
        functools.partial(_fused_body, in_h=in_h, out_h=h, out_w=w, nb=nb,
                          fdim=F),
        grid=(B, nb),
        in_specs=[pl.BlockSpec((in_w, w), lambda b, t: (0, 0)),
                  any_spec, any_spec, feat_spec, feat_spec],
        out_specs=pl.BlockSpec((1, 1), lambda b, t: (0, 0)),
        out_shape=jax.ShapeDtypeStruct((1, 1), jnp.float32),
        scratch_shapes=[
            pltpu.VMEM((5, 2 * RPS, C, in_w), jnp.float32),
            pltpu.VMEM((5, 2 * RPS, C, in_w), jnp.float32),
            pltpu.VMEM((F, NCLS), jnp.float32),
            pltpu.VMEM((8, NCLS), jnp.float32),
            pltpu.VMEM((F, NCLS), jnp.float32),
            pltpu.VMEM((8, NCLS), jnp.float32),
            pltpu.SemaphoreType.DMA((5, 2)),
        ],
    )(wmat, source_softmax, target_softmax, source_feat, target_feat)
    return loss[0, 0]
